# baseline (speedup 1.0000x reference)
import functools

import jax
import jax.numpy as jnp
from jax import lax
from jax.experimental import pallas as pl
from jax.experimental.pallas import tpu as pltpu
from jax.experimental.pallas import tpu_sc as plsc

EPS = 1e-05
L = 16
NCACHE = 0


def _rsqrt_newton(a):
    i = plsc.bitcast(a, jnp.int32)
    i = 0x5F3759DF - (i >> 1)
    y = plsc.bitcast(i, jnp.float32)
    half_a = 0.5 * a
    for _ in range(3):
        y = y * (1.5 - half_a * y * y)
    return y


@functools.partial(jax.jit, static_argnums=(4, 5))
def _cond_io_sc(scale_w, bias_w, xf, lab, n_tokens, h):
    info = plsc.get_sparse_core_info()
    nw = info.num_cores * info.num_subcores
    cb = 16
    tok_per_w = n_tokens // nw
    nchunk = tok_per_w // cb
    nvec = h // L

    mesh = plsc.VectorSubcoreMesh(core_axis_name="c", subcore_axis_name="s")

    @functools.partial(
        pl.kernel,
        out_type=jax.ShapeDtypeStruct((n_tokens, h), jnp.float32),
        mesh=mesh,
        compiler_params=pltpu.CompilerParams(needs_layout_passes=False),
        scratch_types=[
            pltpu.VMEM((tok_per_w,), jnp.int32),
            pltpu.VMEM((4, cb, h), jnp.float32),
            pltpu.VMEM((2, cb, h), jnp.float32),
            pltpu.VMEM((2, cb, h), jnp.float32),
            [pltpu.SemaphoreType.DMA] * 4,
            [pltpu.SemaphoreType.DMA] * 4,
        ],
    )
    def k(scale_hbm, bias_hbm, x_hbm, lab_hbm, out_hbm,
          idx_all, xb, sb, bb, sems_in, sems_out):
        wid = lax.axis_index("s") * info.num_cores + lax.axis_index("c")
        tok0 = wid * tok_per_w

        def in_copies(c, ps, px):
            base = tok0 + c * cb
            idx_sl = idx_all.at[pl.ds(c * cb, cb)]
            return (
                pltpu.make_async_copy(scale_hbm.at[idx_sl], sb.at[ps],
                                      sems_in[px]),
                pltpu.make_async_copy(bias_hbm.at[idx_sl], bb.at[ps],
                                      sems_in[px]),
                pltpu.make_async_copy(x_hbm.at[pl.ds(base, cb)], xb.at[px],
                                      sems_in[px]),
            )

        def in_start(c, ps, px):
            for cp in in_copies(c, ps, px):
                cp.start()

        def in_wait(c, ps, px):
            for cp in in_copies(c, ps, px):
                cp.wait()

        def out_copy(c, px):
            base = tok0 + c * cb
            return pltpu.make_async_copy(xb.at[px],
                                         out_hbm.at[pl.ds(base, cb)],
                                         sems_out[px])

        def compute_chunk(c, ps, px):
            @plsc.parallel_loop(0, cb, unroll=1)
            def tok_body(t):
                acc = [jnp.zeros((L,), jnp.float32) for _ in range(4)]
                asq = [jnp.zeros((L,), jnp.float32) for _ in range(4)]
                xs = []
                for j in range(nvec):
                    v = xb[px, t, pl.ds(j * L, L)]
                    if j < NCACHE:
                        xs.append(v)
                    acc[j % 4] = acc[j % 4] + v
                    asq[j % 4] = asq[j % 4] + v * v
                s1 = jnp.sum((acc[0] + acc[1]) + (acc[2] + acc[3]))
                s2 = jnp.sum((asq[0] + asq[1]) + (asq[2] + asq[3]))
                mean = s1 * (1.0 / h)
                var = s2 * (1.0 / h) - mean * mean
                inv = _rsqrt_newton(jnp.full((L,), var + EPS, jnp.float32))
                mean_v = jnp.full((L,), mean, jnp.float32)
                for j in range(nvec):
                    sl = pl.ds(j * L, L)
                    xv = xs[j] if j < NCACHE else xb[px, t, sl]
                    xb[px, t, sl] = bb[ps, t, sl] + (1.0 + sb[ps, t, sl]) * (
                        (xv - mean_v) * inv)

            out_copy(c, px).start()

        pltpu.sync_copy(lab_hbm.at[pl.ds(tok0, tok_per_w)], idx_all)

        in_start(0, 0, 0)

        def quad_body(i, _):
            c0 = i * 4
            nz = i > 0

            def prefetch(j, guard):
                if guard:
                    @pl.when(nz)
                    def _():
                        out_copy(c0 + j - 4, j).wait()
                else:
                    out_copy(c0 + j - 4, j).wait()
                in_start(c0 + j, j % 2, j)

            prefetch(1, guard=True)
            in_wait(c0, 0, 0)
            compute_chunk(c0, 0, 0)
            prefetch(2, guard=True)
            in_wait(c0 + 1, 1, 1)
            compute_chunk(c0 + 1, 1, 1)
            prefetch(3, guard=True)
            in_wait(c0 + 2, 0, 2)
            compute_chunk(c0 + 2, 0, 2)
            out_copy(c0, 0).wait()
            cnext = jnp.minimum(c0 + 4, nchunk - 1)
            in_start(cnext, 0, 0)
            in_wait(c0 + 3, 1, 3)
            compute_chunk(c0 + 3, 1, 3)
            return 0

        lax.fori_loop(0, nchunk // 4, quad_body, 0)

        in_wait(nchunk - 1, 0, 0)
        for j in range(1, 4):
            out_copy(nchunk - 4 + j, j).wait()

    return k(scale_w, bias_w, xf, lab)


def kernel(x, labels, scale_w, bias_w):
    b, s, h = x.shape
    n = b * s
    xf = x.reshape(n, h)
    lab = labels.reshape(n).astype(jnp.int32)
    out = _cond_io_sc(scale_w, bias_w, xf, lab, n, h)
    return out.reshape(b, s, h)


# speedup vs baseline: 1.2966x; 1.2966x over previous
import functools

import jax
import jax.numpy as jnp
from jax import lax
from jax.experimental import pallas as pl
from jax.experimental.pallas import tpu as pltpu
from jax.experimental.pallas import tpu_sc as plsc

EPS = 1e-05
L = 16
NCACHE = 0


def _rsqrt_newton(a):
    i = plsc.bitcast(a, jnp.int32)
    i = 0x5F3759DF - (i >> 1)
    y = plsc.bitcast(i, jnp.float32)
    half_a = 0.5 * a
    for _ in range(3):
        y = y * (1.5 - half_a * y * y)
    return y


@functools.partial(jax.jit, static_argnums=(4, 5))
def _cond_io_sc(scale_w, bias_w, xf, lab, n_tokens, h):
    info = plsc.get_sparse_core_info()
    nw = info.num_cores * info.num_subcores
    cb = 16
    tok_per_w = n_tokens // nw
    nchunk = tok_per_w // cb
    nvec = h // L

    mesh = plsc.VectorSubcoreMesh(core_axis_name="c", subcore_axis_name="s")

    @functools.partial(
        pl.kernel,
        out_type=jax.ShapeDtypeStruct((n_tokens, h), jnp.float32),
        mesh=mesh,
        compiler_params=pltpu.CompilerParams(needs_layout_passes=False),
        scratch_types=[
            pltpu.VMEM((tok_per_w,), jnp.int32),
            pltpu.VMEM((4, cb, h), jnp.float32),
            pltpu.VMEM((2, cb, h), jnp.float32),
            pltpu.VMEM((2, cb, h), jnp.float32),
            [pltpu.SemaphoreType.DMA] * 4,
            [pltpu.SemaphoreType.DMA] * 4,
        ],
    )
    def k(scale_hbm, bias_hbm, x_hbm, lab_hbm, out_hbm,
          idx_all, xb, sb, bb, sems_in, sems_out):
        wid = lax.axis_index("s") * info.num_cores + lax.axis_index("c")
        tok0 = wid * tok_per_w

        def in_copies(c, ps, px):
            base = tok0 + c * cb
            idx_sl = idx_all.at[pl.ds(c * cb, cb)]
            return (
                pltpu.make_async_copy(scale_hbm.at[idx_sl], sb.at[ps],
                                      sems_in[px]),
                pltpu.make_async_copy(bias_hbm.at[idx_sl], bb.at[ps],
                                      sems_in[px]),
                pltpu.make_async_copy(x_hbm.at[pl.ds(base, cb)], xb.at[px],
                                      sems_in[px]),
            )

        def in_start(c, ps, px):
            for cp in in_copies(c, ps, px):
                cp.start()

        def in_wait(c, ps, px):
            for cp in in_copies(c, ps, px):
                cp.wait()

        def out_copy(c, px):
            base = tok0 + c * cb
            return pltpu.make_async_copy(xb.at[px],
                                         out_hbm.at[pl.ds(base, cb)],
                                         sems_out[px])

        def compute_chunk(c, ps, px):
            @plsc.parallel_loop(0, cb, unroll=2)
            def tok_body(t):
                acc = [jnp.zeros((L,), jnp.float32) for _ in range(4)]
                asq = [jnp.zeros((L,), jnp.float32) for _ in range(4)]
                xs = []
                for j in range(nvec):
                    v = xb[px, t, pl.ds(j * L, L)]
                    if j < NCACHE:
                        xs.append(v)
                    acc[j % 4] = acc[j % 4] + v
                    asq[j % 4] = asq[j % 4] + v * v
                s1 = jnp.sum((acc[0] + acc[1]) + (acc[2] + acc[3]))
                s2 = jnp.sum((asq[0] + asq[1]) + (asq[2] + asq[3]))
                mean = s1 * (1.0 / h)
                var = s2 * (1.0 / h) - mean * mean
                inv = _rsqrt_newton(jnp.full((L,), var + EPS, jnp.float32))
                mean_v = jnp.full((L,), mean, jnp.float32)
                for j in range(nvec):
                    sl = pl.ds(j * L, L)
                    xv = xs[j] if j < NCACHE else xb[px, t, sl]
                    xb[px, t, sl] = bb[ps, t, sl] + (1.0 + sb[ps, t, sl]) * (
                        (xv - mean_v) * inv)

            out_copy(c, px).start()

        pltpu.sync_copy(lab_hbm.at[pl.ds(tok0, tok_per_w)], idx_all)

        in_start(0, 0, 0)

        def quad_body(i, _):
            c0 = i * 4
            nz = i > 0

            def prefetch(j, guard):
                if guard:
                    @pl.when(nz)
                    def _():
                        out_copy(c0 + j - 4, j).wait()
                else:
                    out_copy(c0 + j - 4, j).wait()
                in_start(c0 + j, j % 2, j)

            prefetch(1, guard=True)
            in_wait(c0, 0, 0)
            compute_chunk(c0, 0, 0)
            prefetch(2, guard=True)
            in_wait(c0 + 1, 1, 1)
            compute_chunk(c0 + 1, 1, 1)
            prefetch(3, guard=True)
            in_wait(c0 + 2, 0, 2)
            compute_chunk(c0 + 2, 0, 2)
            out_copy(c0, 0).wait()
            cnext = jnp.minimum(c0 + 4, nchunk - 1)
            in_start(cnext, 0, 0)
            in_wait(c0 + 3, 1, 3)
            compute_chunk(c0 + 3, 1, 3)
            return 0

        lax.fori_loop(0, nchunk // 4, quad_body, 0)

        in_wait(nchunk - 1, 0, 0)
        for j in range(1, 4):
            out_copy(nchunk - 4 + j, j).wait()

    return k(scale_w, bias_w, xf, lab)


def kernel(x, labels, scale_w, bias_w):
    b, s, h = x.shape
    n = b * s
    xf = x.reshape(n, h)
    lab = labels.reshape(n).astype(jnp.int32)
    out = _cond_io_sc(scale_w, bias_w, xf, lab, n, h)
    return out.reshape(b, s, h)
